# baseline (speedup 1.0000x reference)
import functools

import numpy as np
import jax
import jax.numpy as jnp
from jax import lax
from jax.experimental import pallas as pl
from jax.experimental.pallas import tpu as pltpu
from jax.experimental.pallas import tpu_sc as plsc

_B = 128
_C = 5
_S = 256
_H = 128
_L = 16
_NC, _NS = 2, 16
_NW = _NC * _NS
_RCH = 32
_CPT = _H // _RCH
_CPW = (_B * 2 * _CPT) // _NW
_HSLOTS = 2 * _L * 8


def _sc_hist_body(seg_hbm, out_hbm, buf, hist, sem0, sem1):
    wid = lax.axis_index("s") * _NC + lax.axis_index("c")
    g0 = wid * _CPW
    lane = lax.iota(jnp.int32, _L)
    bas = (lane * 16, lane * 16 + 8)
    ones = jnp.ones((_L,), jnp.float32)
    zf = jnp.zeros((_L,), jnp.float32)
    sems = (sem0, sem1)

    def clear_hist():
        for i in range(_HSLOTS // _L):
            hist[pl.ds(i * _L, _L)] = zf

    def chunk_addr(gg):
        b = gg >> 3
        tb = (gg >> 2) & 1
        r0 = tb * _H + (gg & 3) * _RCH
        return b, tb, r0

    def start(gg, slot):
        b, _, r0 = chunk_addr(gg)
        for c in range(_C):
            pltpu.async_copy(
                seg_hbm.at[b, c, pl.ds(r0, _RCH), :], buf.at[slot, c], sems[slot])

    def wait(gg, slot):
        b, _, r0 = chunk_addr(gg)
        for c in range(_C):
            pltpu.make_async_copy(
                seg_hbm.at[b, c, pl.ds(r0, _RCH), :], buf.at[slot, c],
                sems[slot]).wait()

    def compute(slot):
        def row(r, carry):
            for lr in range(2):
                for j in range(8):
                    col = lr * _H + j * _L
                    bv = buf[slot, 0, r, pl.ds(col, _L)]
                    bi = jnp.zeros((_L,), jnp.int32)
                    for c in range(1, _C):
                        xc = buf[slot, c, r, pl.ds(col, _L)]
                        gt = xc > bv
                        bv = jnp.where(gt, xc, bv)
                        bi = jnp.where(gt, jnp.int32(c), bi)
                    plsc.addupdate_scatter(hist, [bas[lr] + bi], ones)
            return carry
        lax.fori_loop(0, _RCH, row, 0)

    clear_hist()
    start(g0, 0)
    start(g0 + 1, 1)

    def outer(gp, carry):
        for slot in range(2):
            g = gp * 2 + slot
            gg = g0 + g
            wait(gg, slot)
            compute(slot)

            @pl.when(g + 2 < _CPW)
            def _():
                start(gg + 2, slot)

            b, tb, _ = chunk_addr(gg)

            @pl.when((gg & 3) == 3)
            def _():
                pltpu.sync_copy(hist, out_hbm.at[b, pl.ds(tb * _HSLOTS, _HSLOTS)])
                clear_hist()
        return carry

    lax.fori_loop(0, _CPW // 2, outer, 0)


@functools.cache
def _get_sc_hist():
    return pl.kernel(
        _sc_hist_body,
        out_type=jax.ShapeDtypeStruct((_B, 2 * _HSLOTS), jnp.float32),
        mesh=plsc.VectorSubcoreMesh(core_axis_name="c", subcore_axis_name="s"),
        scratch_types=[
            pltpu.VMEM((2, _C, _RCH, _S), jnp.float32),
            pltpu.VMEM((_HSLOTS,), jnp.float32),
            pltpu.SemaphoreType.DMA,
            pltpu.SemaphoreType.DMA,
        ],
        compiler_params=pltpu.CompilerParams(needs_layout_passes=False),
    )


def _build_selectors():
    g5 = np.zeros((2 * _HSLOTS, _C), np.float32)
    g20 = np.zeros((2 * _HSLOTS, 4 * _C), np.float32)
    for tb in range(2):
        for ln in range(_L):
            for lr in range(2):
                for c in range(_C):
                    j = tb * _HSLOTS + ln * 16 + lr * 8 + c
                    g5[j, c] = 1.0 / float(_S * _S)
                    g20[j, (lr * 2 + tb) * _C + c] = 1.0 / float(_H * _H)
    return g5, g20


_G5, _G20 = _build_selectors()


def _stage2_body(y_ref, g5_ref, g20_ref, m5_ref, iv5_ref, m20_ref, iv20_ref,
                 s_ref, o_ref):
    y = y_ref[...]
    u5 = jnp.dot(y, g5_ref[...], preferred_element_type=jnp.float32)
    u20 = jnp.dot(y, g20_ref[...], preferred_element_type=jnp.float32)
    d5 = u5 - m5_ref[...]
    d20 = u20 - m20_ref[...]
    q5 = jnp.sum(d5 * jnp.dot(d5, iv5_ref[...], preferred_element_type=jnp.float32),
                 axis=1, keepdims=True)
    q20 = jnp.sum(d20 * jnp.dot(d20, iv20_ref[...], preferred_element_type=jnp.float32),
                  axis=1, keepdims=True)
    hvm = s_ref[:, 0:1]
    hvs = s_ref[:, 1:2]
    phvm = s_ref[:, 2:3]
    phvs = s_ref[:, 3:4]
    o_ref[...] = (jnp.sqrt(q5) - hvm) / hvs + (jnp.sqrt(q20) - phvm) / phvs


_stage2 = pl.pallas_call(
    _stage2_body,
    out_shape=jax.ShapeDtypeStruct((_B, 1), jnp.float32),
)


def kernel(segmap, hist_mean, hist_invcov, patch_hist_mean, patch_hist_invcov,
           hist_val_mean, hist_val_std, patch_hist_val_mean, patch_hist_val_std):
    raw = _get_sc_hist()(segmap)
    s = jnp.stack([hist_val_mean, hist_val_std,
                   patch_hist_val_mean, patch_hist_val_std]).reshape(1, 4)
    out = _stage2(raw, jnp.asarray(_G5), jnp.asarray(_G20),
                  hist_mean.reshape(1, _C), hist_invcov,
                  patch_hist_mean.reshape(1, 4 * _C), patch_hist_invcov, s)
    return out.reshape(_B)


# speedup vs baseline: 2.6071x; 2.6071x over previous
import functools

import numpy as np
import jax
import jax.numpy as jnp
from jax import lax
from jax.experimental import pallas as pl
from jax.experimental.pallas import tpu as pltpu
from jax.experimental.pallas import tpu_sc as plsc

_B = 128
_C = 5
_S = 256
_H = 128
_L = 16
_NC, _NS = 2, 16
_NW = _NC * _NS
_RCH = 32
_CPT = _H // _RCH
_CPW = (_B * 2 * _CPT) // _NW
_HSLOTS = 8 * _L


def _sc_hist_body(seg_hbm, out_hbm, buf, hist, sem0, sem1):
    wid = lax.axis_index("s") * _NC + lax.axis_index("c")
    g0 = wid * _CPW
    one = jnp.ones((_L,), jnp.float32)
    zf = jnp.zeros((_L,), jnp.float32)
    sems = (sem0, sem1)

    def chunk_addr(gg):
        b = gg >> 3
        tb = (gg >> 2) & 1
        r0 = tb * _H + (gg & 3) * _RCH
        return b, tb, r0

    def start(gg, slot):
        b, _, r0 = chunk_addr(gg)
        for c in range(_C):
            pltpu.async_copy(
                seg_hbm.at[b, c, pl.ds(r0, _RCH), :], buf.at[slot, c], sems[slot])

    def wait(gg, slot):
        b, _, r0 = chunk_addr(gg)
        for c in range(_C):
            pltpu.make_async_copy(
                seg_hbm.at[b, c, pl.ds(r0, _RCH), :], buf.at[slot, c],
                sems[slot]).wait()

    def compute(slot, accs):
        def sel4(r, col):
            xs = [buf[slot, c, r, pl.ds(col, _L)] for c in range(_C)]
            m = jnp.maximum(
                jnp.maximum(jnp.maximum(xs[0], xs[1]),
                            jnp.maximum(xs[2], xs[3])), xs[4])
            return [jnp.where(xs[cp + 1] == m, one, zf) for cp in range(4)]

        def half(lr):
            def row(r, a4):
                a4 = list(a4)
                for jp in range(4):
                    s_even = sel4(r, lr * _H + (2 * jp) * _L)
                    s_odd = sel4(r, lr * _H + (2 * jp + 1) * _L)
                    for cp in range(4):
                        a4[cp] = a4[cp] + (s_even[cp] + s_odd[cp])
                return tuple(a4)
            return row

        accl = lax.fori_loop(0, _RCH, half(0), tuple(accs[0:4]))
        accr = lax.fori_loop(0, _RCH, half(1), tuple(accs[4:8]))
        return accl + accr

    start(g0, 0)
    start(g0 + 1, 1)

    def outer(gp, accs):
        for slot in range(2):
            g = gp * 2 + slot
            gg = g0 + g
            wait(gg, slot)
            accs = compute(slot, accs)

            @pl.when(g + 2 < _CPW)
            def _():
                start(gg + 2, slot)

            b, tb, _ = chunk_addr(gg)
            flush = (gg & 3) == 3

            @pl.when(flush)
            def _():
                for k in range(8):
                    hist[pl.ds(k * _L, _L)] = accs[k]
                pltpu.sync_copy(hist, out_hbm.at[b, pl.ds(tb * _HSLOTS, _HSLOTS)])

            accs = tuple(jnp.where(flush, zf, a) for a in accs)
        return accs

    lax.fori_loop(0, _CPW // 2, outer, tuple(zf for _ in range(8)))


@functools.cache
def _get_sc_hist():
    return pl.kernel(
        _sc_hist_body,
        out_type=jax.ShapeDtypeStruct((_B, 2 * _HSLOTS), jnp.float32),
        mesh=plsc.VectorSubcoreMesh(core_axis_name="c", subcore_axis_name="s"),
        scratch_types=[
            pltpu.VMEM((2, _C, _RCH, _S), jnp.float32),
            pltpu.VMEM((_HSLOTS,), jnp.float32),
            pltpu.SemaphoreType.DMA,
            pltpu.SemaphoreType.DMA,
        ],
        compiler_params=pltpu.CompilerParams(needs_layout_passes=False),
    )


def _build_selectors():
    g5 = np.zeros((2 * _HSLOTS, _C), np.float32)
    g20 = np.zeros((2 * _HSLOTS, 4 * _C), np.float32)
    b5 = np.zeros((1, _C), np.float32)
    b20 = np.zeros((1, 4 * _C), np.float32)
    b5[0, 0] = 1.0
    for tb in range(2):
        for lr in range(2):
            q = lr * 2 + tb
            b20[0, q * _C] = 1.0
            for cp in range(_C - 1):
                for ln in range(_L):
                    j = tb * _HSLOTS + (lr * 4 + cp) * _L + ln
                    g5[j, cp + 1] = 1.0 / float(_S * _S)
                    g5[j, 0] = -1.0 / float(_S * _S)
                    g20[j, q * _C + cp + 1] = 1.0 / float(_H * _H)
                    g20[j, q * _C] = -1.0 / float(_H * _H)
    return g5, g20, b5, b20


_G5, _G20, _B5, _B20 = _build_selectors()


def _stage2_body(y_ref, g5_ref, g20_ref, b5_ref, b20_ref, m5_ref, iv5_ref,
                 m20_ref, iv20_ref, s_ref, o_ref):
    y = y_ref[...]
    u5 = jnp.dot(y, g5_ref[...], preferred_element_type=jnp.float32)
    u20 = jnp.dot(y, g20_ref[...], preferred_element_type=jnp.float32)
    d5 = u5 + (b5_ref[...] - m5_ref[...])
    d20 = u20 + (b20_ref[...] - m20_ref[...])
    q5 = jnp.sum(d5 * jnp.dot(d5, iv5_ref[...], preferred_element_type=jnp.float32),
                 axis=1, keepdims=True)
    q20 = jnp.sum(d20 * jnp.dot(d20, iv20_ref[...], preferred_element_type=jnp.float32),
                  axis=1, keepdims=True)
    hvm = s_ref[:, 0:1]
    hvs = s_ref[:, 1:2]
    phvm = s_ref[:, 2:3]
    phvs = s_ref[:, 3:4]
    o_ref[...] = (jnp.sqrt(q5) - hvm) / hvs + (jnp.sqrt(q20) - phvm) / phvs


_stage2 = pl.pallas_call(
    _stage2_body,
    out_shape=jax.ShapeDtypeStruct((_B, 1), jnp.float32),
)


def kernel(segmap, hist_mean, hist_invcov, patch_hist_mean, patch_hist_invcov,
           hist_val_mean, hist_val_std, patch_hist_val_mean, patch_hist_val_std):
    raw = _get_sc_hist()(segmap)
    s = jnp.stack([hist_val_mean, hist_val_std,
                   patch_hist_val_mean, patch_hist_val_std]).reshape(1, 4)
    out = _stage2(raw, jnp.asarray(_G5), jnp.asarray(_G20),
                  jnp.asarray(_B5), jnp.asarray(_B20),
                  hist_mean.reshape(1, _C), hist_invcov,
                  patch_hist_mean.reshape(1, 4 * _C), patch_hist_invcov, s)
    return out.reshape(_B)


# speedup vs baseline: 2.6118x; 1.0018x over previous
import functools

import numpy as np
import jax
import jax.numpy as jnp
from jax import lax
from jax.experimental import pallas as pl
from jax.experimental.pallas import tpu as pltpu
from jax.experimental.pallas import tpu_sc as plsc

_B = 128
_C = 5
_S = 256
_H = 128
_L = 16
_NC, _NS = 2, 16
_NW = _NC * _NS
_RCH = 32
_CPT = _H // _RCH
_CPW = (_B * 2 * _CPT) // _NW
_HSLOTS = 8 * _L


def _sc_hist_body(seg_hbm, out_hbm, buf, hist, sem0, sem1):
    wid = lax.axis_index("s") * _NC + lax.axis_index("c")
    g0 = wid * _CPW
    one = jnp.ones((_L,), jnp.float32)
    zf = jnp.zeros((_L,), jnp.float32)
    sems = (sem0, sem1)

    def chunk_addr(gg):
        b = gg >> 3
        tb = (gg >> 2) & 1
        r0 = tb * _H + (gg & 3) * _RCH
        return b, tb, r0

    def start(gg, slot):
        b, _, r0 = chunk_addr(gg)
        pltpu.async_copy(
            seg_hbm.at[b, :, pl.ds(r0, _RCH), :], buf.at[slot], sems[slot])

    def wait(gg, slot):
        b, _, r0 = chunk_addr(gg)
        pltpu.make_async_copy(
            seg_hbm.at[b, :, pl.ds(r0, _RCH), :], buf.at[slot],
            sems[slot]).wait()

    def compute(slot, accs):
        def sel4(r, col):
            xs = [buf[slot, c, r, pl.ds(col, _L)] for c in range(_C)]
            m = jnp.maximum(
                jnp.maximum(jnp.maximum(xs[0], xs[1]),
                            jnp.maximum(xs[2], xs[3])), xs[4])
            return [jnp.where(xs[cp + 1] == m, one, zf) for cp in range(4)]

        def half(lr):
            def row(r, a4):
                a4 = list(a4)
                for jp in range(4):
                    s_even = sel4(r, lr * _H + (2 * jp) * _L)
                    s_odd = sel4(r, lr * _H + (2 * jp + 1) * _L)
                    for cp in range(4):
                        a4[cp] = a4[cp] + (s_even[cp] + s_odd[cp])
                return tuple(a4)
            return row

        accl = lax.fori_loop(0, _RCH, half(0), tuple(accs[0:4]))
        accr = lax.fori_loop(0, _RCH, half(1), tuple(accs[4:8]))
        return accl + accr

    start(g0, 0)
    start(g0 + 1, 1)

    def outer(gp, accs):
        for slot in range(2):
            g = gp * 2 + slot
            gg = g0 + g
            wait(gg, slot)
            accs = compute(slot, accs)

            @pl.when(g + 2 < _CPW)
            def _():
                start(gg + 2, slot)

            b, tb, _ = chunk_addr(gg)
            flush = (gg & 3) == 3

            @pl.when(flush)
            def _():
                for k in range(8):
                    hist[pl.ds(k * _L, _L)] = accs[k]
                pltpu.sync_copy(hist, out_hbm.at[b, pl.ds(tb * _HSLOTS, _HSLOTS)])

            accs = tuple(jnp.where(flush, zf, a) for a in accs)
        return accs

    lax.fori_loop(0, _CPW // 2, outer, tuple(zf for _ in range(8)))


@functools.cache
def _get_sc_hist():
    return pl.kernel(
        _sc_hist_body,
        out_type=jax.ShapeDtypeStruct((_B, 2 * _HSLOTS), jnp.float32),
        mesh=plsc.VectorSubcoreMesh(core_axis_name="c", subcore_axis_name="s"),
        scratch_types=[
            pltpu.VMEM((2, _C, _RCH, _S), jnp.float32),
            pltpu.VMEM((_HSLOTS,), jnp.float32),
            pltpu.SemaphoreType.DMA,
            pltpu.SemaphoreType.DMA,
        ],
        compiler_params=pltpu.CompilerParams(needs_layout_passes=False),
    )


def _build_selectors():
    g5 = np.zeros((2 * _HSLOTS, _C), np.float32)
    g20 = np.zeros((2 * _HSLOTS, 4 * _C), np.float32)
    b5 = np.zeros((1, _C), np.float32)
    b20 = np.zeros((1, 4 * _C), np.float32)
    b5[0, 0] = 1.0
    for tb in range(2):
        for lr in range(2):
            q = lr * 2 + tb
            b20[0, q * _C] = 1.0
            for cp in range(_C - 1):
                for ln in range(_L):
                    j = tb * _HSLOTS + (lr * 4 + cp) * _L + ln
                    g5[j, cp + 1] = 1.0 / float(_S * _S)
                    g5[j, 0] = -1.0 / float(_S * _S)
                    g20[j, q * _C + cp + 1] = 1.0 / float(_H * _H)
                    g20[j, q * _C] = -1.0 / float(_H * _H)
    return g5, g20, b5, b20


_G5, _G20, _B5, _B20 = _build_selectors()


def _stage2_body(y_ref, g5_ref, g20_ref, b5_ref, b20_ref, m5_ref, iv5_ref,
                 m20_ref, iv20_ref, s_ref, o_ref):
    y = y_ref[...]
    u5 = jnp.dot(y, g5_ref[...], preferred_element_type=jnp.float32)
    u20 = jnp.dot(y, g20_ref[...], preferred_element_type=jnp.float32)
    d5 = u5 + (b5_ref[...] - m5_ref[...])
    d20 = u20 + (b20_ref[...] - m20_ref[...])
    q5 = jnp.sum(d5 * jnp.dot(d5, iv5_ref[...], preferred_element_type=jnp.float32),
                 axis=1, keepdims=True)
    q20 = jnp.sum(d20 * jnp.dot(d20, iv20_ref[...], preferred_element_type=jnp.float32),
                  axis=1, keepdims=True)
    hvm = s_ref[:, 0:1]
    hvs = s_ref[:, 1:2]
    phvm = s_ref[:, 2:3]
    phvs = s_ref[:, 3:4]
    o_ref[...] = (jnp.sqrt(q5) - hvm) / hvs + (jnp.sqrt(q20) - phvm) / phvs


_stage2 = pl.pallas_call(
    _stage2_body,
    out_shape=jax.ShapeDtypeStruct((_B, 1), jnp.float32),
)


def kernel(segmap, hist_mean, hist_invcov, patch_hist_mean, patch_hist_invcov,
           hist_val_mean, hist_val_std, patch_hist_val_mean, patch_hist_val_std):
    raw = _get_sc_hist()(segmap)
    s = jnp.stack([hist_val_mean, hist_val_std,
                   patch_hist_val_mean, patch_hist_val_std]).reshape(1, 4)
    out = _stage2(raw, jnp.asarray(_G5), jnp.asarray(_G20),
                  jnp.asarray(_B5), jnp.asarray(_B20),
                  hist_mean.reshape(1, _C), hist_invcov,
                  patch_hist_mean.reshape(1, 4 * _C), patch_hist_invcov, s)
    return out.reshape(_B)
